# baseline (speedup 1.0000x reference)
import functools

import jax
import jax.numpy as jnp
from jax import lax
from jax.experimental import pallas as pl
from jax.experimental.pallas import tpu as pltpu
from jax.experimental.pallas import tpu_sc as plsc

L = 16
DIM = 64
CHUNK = 128


def _shuffle(v, p):
    dnums = lax.GatherDimensionNumbers(
        offset_dims=(), collapsed_slice_dims=(0,), start_index_map=(0,))
    return lax.gather(v, p[:, None], dnums, slice_sizes=(1,),
                      mode=lax.GatherScatterMode.PROMISE_IN_BOUNDS)


def _neg_sqrt(x):
    xc = jnp.maximum(x, jnp.float32(1e-30))
    i = lax.bitcast_convert_type(xc, jnp.int32)
    i = jnp.int32(0x5F3759DF) - lax.shift_right_arithmetic(i, jnp.int32(1))
    y = lax.bitcast_convert_type(i, jnp.float32)
    for _ in range(3):
        y = y * (jnp.float32(1.5) - jnp.float32(0.5) * xc * y * y)
    return -(xc * y)


def kernel(h_idx, r_idx, t_idx, entity_emb, relation_emb):
    B = h_idx.shape[0]
    info = plsc.get_sparse_core_info()
    NC, NS = info.num_cores, info.num_subcores
    NW = NC * NS
    W = B // NW
    NCH = W // CHUNK
    GROUPS = W // L

    mesh = plsc.VectorSubcoreMesh(core_axis_name="c", subcore_axis_name="s")

    @functools.partial(
        pl.kernel,
        out_type=jax.ShapeDtypeStruct((B,), jnp.float32),
        mesh=mesh,
        compiler_params=pltpu.CompilerParams(use_tc_tiling_on_sc=False),
        scratch_types=[
            pltpu.VMEM((NCH, CHUNK), jnp.int32),
            pltpu.VMEM((NCH, CHUNK), jnp.int32),
            pltpu.VMEM((NCH, CHUNK), jnp.int32),
            pltpu.VMEM((W, DIM), jnp.float32),
            pltpu.VMEM((W, DIM), jnp.float32),
            pltpu.VMEM((W, DIM), jnp.float32),
            pltpu.VMEM((W,), jnp.float32),
            pltpu.SemaphoreType.DMA,
        ],
    )
    def trans_e(h_idx_hbm, r_idx_hbm, t_idx_hbm, ent_hbm, rel_hbm, out_hbm,
                hi_v, ri_v, ti_v, h_v, r_v, t_v, out_v, sem):
        wid = lax.axis_index("s") * NC + lax.axis_index("c")
        base = wid * W

        for j in range(NCH):
            off = base + j * CHUNK
            pltpu.sync_copy(h_idx_hbm.at[pl.ds(off, CHUNK)], hi_v.at[j])
            pltpu.sync_copy(r_idx_hbm.at[pl.ds(off, CHUNK)], ri_v.at[j])
            pltpu.sync_copy(t_idx_hbm.at[pl.ds(off, CHUNK)], ti_v.at[j])

        copies = []
        for j in range(NCH):
            d = pl.ds(j * CHUNK, CHUNK)
            copies.append(pltpu.async_copy(ent_hbm.at[hi_v.at[j]], h_v.at[d], sem))
            copies.append(pltpu.async_copy(rel_hbm.at[ri_v.at[j]], r_v.at[d], sem))
            copies.append(pltpu.async_copy(ent_hbm.at[ti_v.at[j]], t_v.at[d], sem))
        for c in copies:
            c.wait()

        lane = lax.iota(jnp.int32, L)
        perms = [lane ^ s for s in (8, 4, 2, 1)]

        def group(g, carry):
            svec = jnp.zeros((L,), jnp.float32)
            for i in range(L):
                row = g * L + i
                acc = jnp.zeros((L,), jnp.float32)
                for c4 in range(DIM // L):
                    ds = pl.ds(c4 * L, L)
                    dlt = h_v[row, ds] + r_v[row, ds] - t_v[row, ds]
                    acc = acc + dlt * dlt
                for p in perms:
                    acc = acc + _shuffle(acc, p)
                svec = jnp.where(lane == i, acc, svec)
            out_v[pl.ds(g * L, L)] = _neg_sqrt(svec)
            return carry

        lax.fori_loop(0, GROUPS, group, jnp.int32(0))
        pltpu.sync_copy(out_v, out_hbm.at[pl.ds(base, W)])

    return trans_e(h_idx.astype(jnp.int32), r_idx.astype(jnp.int32),
                   t_idx.astype(jnp.int32), entity_emb, relation_emb)


# speedup vs baseline: 1.6774x; 1.6774x over previous
import functools

import jax
import jax.numpy as jnp
from jax import lax
from jax.experimental import pallas as pl
from jax.experimental.pallas import tpu as pltpu
from jax.experimental.pallas import tpu_sc as plsc

L = 16
DIM = 64


def _shuffle(v, p):
    dnums = lax.GatherDimensionNumbers(
        offset_dims=(), collapsed_slice_dims=(0,), start_index_map=(0,))
    return lax.gather(v, p[:, None], dnums, slice_sizes=(1,),
                      mode=lax.GatherScatterMode.PROMISE_IN_BOUNDS)


def _neg_sqrt(x):
    xc = jnp.maximum(x, jnp.float32(1e-30))
    i = lax.bitcast_convert_type(xc, jnp.int32)
    i = jnp.int32(0x5F3759DF) - lax.shift_right_arithmetic(i, jnp.int32(1))
    y = lax.bitcast_convert_type(i, jnp.float32)
    for _ in range(3):
        y = y * (jnp.float32(1.5) - jnp.float32(0.5) * xc * y * y)
    return -(xc * y)


def kernel(h_idx, r_idx, t_idx, entity_emb, relation_emb):
    B = h_idx.shape[0]
    info = plsc.get_sparse_core_info()
    NC, NS = info.num_cores, info.num_subcores
    NW = NC * NS
    W = B // NW
    NGR = W // L

    mesh = plsc.VectorSubcoreMesh(core_axis_name="c", subcore_axis_name="s")

    @functools.partial(
        pl.kernel,
        out_type=jax.ShapeDtypeStruct((B,), jnp.float32),
        mesh=mesh,
        scratch_types=[
            pltpu.VMEM((W,), jnp.int32),
            pltpu.VMEM((W,), jnp.int32),
            pltpu.VMEM((W,), jnp.int32),
            pltpu.VMEM((2, L, DIM), jnp.float32),
            pltpu.VMEM((2, L, DIM), jnp.float32),
            pltpu.VMEM((2, L, DIM), jnp.float32),
            pltpu.VMEM((W,), jnp.float32),
            pltpu.SemaphoreType.DMA,
            pltpu.SemaphoreType.DMA,
        ],
    )
    def trans_e(h_idx_hbm, r_idx_hbm, t_idx_hbm, ent_hbm, rel_hbm, out_hbm,
                hi_v, ri_v, ti_v, h_v, r_v, t_v, out_v, sem0, sem1):
        wid = lax.axis_index("s") * NC + lax.axis_index("c")
        base = wid * W

        pltpu.sync_copy(h_idx_hbm.at[pl.ds(base, W)], hi_v)
        pltpu.sync_copy(r_idx_hbm.at[pl.ds(base, W)], ri_v)
        pltpu.sync_copy(t_idx_hbm.at[pl.ds(base, W)], ti_v)

        def fire(g, b, sem):
            eds = pl.ds(g * L, L)
            hvec, rvec, tvec = hi_v[eds], ri_v[eds], ti_v[eds]
            for i in range(L):
                pltpu.async_copy(ent_hbm.at[hvec[i]], h_v.at[b, i], sem)
                pltpu.async_copy(rel_hbm.at[rvec[i]], r_v.at[b, i], sem)
                pltpu.async_copy(ent_hbm.at[tvec[i]], t_v.at[b, i], sem)

        def drain(b, sem):
            for buf in (h_v, r_v, t_v):
                for i in range(L):
                    pltpu.make_async_copy(ent_hbm.at[0], buf.at[b, i], sem).wait()

        lane = lax.iota(jnp.int32, L)
        perms = [lane ^ s for s in (8, 4, 2, 1)]

        def compute(g, b):
            svec = jnp.zeros((L,), jnp.float32)
            for i in range(L):
                acc = jnp.zeros((L,), jnp.float32)
                for c4 in range(DIM // L):
                    ds = pl.ds(c4 * L, L)
                    dlt = h_v[b, i, ds] + r_v[b, i, ds] - t_v[b, i, ds]
                    acc = acc + dlt * dlt
                for p in perms:
                    acc = acc + _shuffle(acc, p)
                svec = jnp.where(lane == i, acc, svec)
            out_v[pl.ds(g * L, L)] = _neg_sqrt(svec)

        fire(0, 0, sem0)

        def pair_step(p, carry):
            g0 = p * 2
            fire(g0 + 1, 1, sem1)
            drain(0, sem0)
            compute(g0, 0)

            @pl.when(p < (NGR // 2 - 1))
            def _():
                fire(g0 + 2, 0, sem0)

            drain(1, sem1)
            compute(g0 + 1, 1)
            return carry

        lax.fori_loop(0, NGR // 2, pair_step, jnp.int32(0))
        pltpu.sync_copy(out_v, out_hbm.at[pl.ds(base, W)])

    return trans_e(h_idx.astype(jnp.int32), r_idx.astype(jnp.int32),
                   t_idx.astype(jnp.int32), entity_emb, relation_emb)
